# baseline (speedup 1.0000x reference)
import functools

import jax
import jax.numpy as jnp
from jax import lax
from jax.experimental import pallas as pl
from jax.experimental.pallas import tpu as pltpu
from jax.experimental.pallas import tpu_sc as plsc

N_NODES = 10000
N_EDGES = 320000
H = 128

NC = 2
NS = 16
NW = NC * NS
EPT = N_EDGES // NW
CHUNK = 80
NCHUNK = EPT // CHUNK
LANES = 16
KV = H // LANES
NBUF = 3

N_PAD = 10240


def _linear_body(x_ref, w_ref, b_ref, o_ref):
    x = x_ref[...]
    w = w_ref[...]
    y = lax.dot_general(x, w, (((1,), (1,)), ((), ())),
                        preferred_element_type=jnp.float32)
    o_ref[...] = y + b_ref[...]


def _linear(x, w, b, tile):
    m, k = x.shape
    out = w.shape[0]
    grid = (m // tile,)
    return pl.pallas_call(
        _linear_body,
        grid=grid,
        in_specs=[
            pl.BlockSpec((tile, k), lambda i: (i, 0)),
            pl.BlockSpec((out, k), lambda i: (0, 0)),
            pl.BlockSpec((1, out), lambda i: (0, 0)),
        ],
        out_specs=pl.BlockSpec((tile, out), lambda i: (i, 0)),
        out_shape=jax.ShapeDtypeStruct((m, out), jnp.float32),
    )(x, w, b.reshape(1, out))


def _project_body(x_ref, w_ref, hw_ref):
    hw_ref[...] = lax.dot_general(
        x_ref[...], w_ref[...], (((1,), (1,)), ((), ())),
        preferred_element_type=jnp.float32)


def _project(x, w, tile=2000):
    m, k = x.shape
    out = w.shape[0]
    grid = (m // tile,)
    return pl.pallas_call(
        _project_body,
        grid=grid,
        in_specs=[
            pl.BlockSpec((tile, k), lambda i: (i, 0)),
            pl.BlockSpec((out, k), lambda i: (0, 0)),
        ],
        out_specs=pl.BlockSpec((tile, out), lambda i: (i, 0)),
        out_shape=jax.ShapeDtypeStruct((m, out), jnp.float32),
    )(x, w)


def _node_body(has_next, h_ref, ap_ref, dp_ref, w1h_ref, w1a_ref, b1_ref,
               w2_ref, b2_ref, *rest):
    if has_next:
        wea_ref, hn_ref, hw_ref = rest
    else:
        (hn_ref,) = rest
    deg = dp_ref[0] + dp_ref[1]
    rdeg = 1.0 / jnp.maximum(deg, 1.0)
    agg = (ap_ref[0] + ap_ref[1]) * rdeg
    x1 = lax.dot_general(h_ref[...], w1h_ref[...], (((1,), (1,)), ((), ())),
                         preferred_element_type=jnp.float32)
    x1 = x1 + lax.dot_general(agg, w1a_ref[...], (((1,), (1,)), ((), ())),
                              preferred_element_type=jnp.float32)
    x1 = x1 + b1_ref[...]
    hid = x1 * jax.nn.sigmoid(x1)
    h_new = lax.dot_general(hid, w2_ref[...], (((1,), (1,)), ((), ())),
                            preferred_element_type=jnp.float32)
    h_new = h_new + b2_ref[...]
    hn_ref[...] = h_new
    if has_next:
        hw_ref[...] = lax.dot_general(
            h_new, wea_ref[...], (((1,), (1,)), ((), ())),
            preferred_element_type=jnp.float32)


def _node_update(h, agg_parts, deg_parts, w1h, w1a, b1, w2, b2, wea_next,
                 tile=2000):
    n = h.shape[0]
    grid = (n // tile,)
    has_next = wea_next is not None
    in_specs = [
        pl.BlockSpec((tile, H), lambda i: (i, 0)),
        pl.BlockSpec((2, tile, H), lambda i: (0, i, 0)),
        pl.BlockSpec((2, tile, 1), lambda i: (0, i, 0)),
        pl.BlockSpec((H, H), lambda i: (0, 0)),
        pl.BlockSpec((H, H), lambda i: (0, 0)),
        pl.BlockSpec((1, H), lambda i: (0, 0)),
        pl.BlockSpec((H, H), lambda i: (0, 0)),
        pl.BlockSpec((1, H), lambda i: (0, 0)),
    ]
    args = [h, agg_parts, deg_parts, w1h, w1a, b1.reshape(1, H), w2,
            b2.reshape(1, H)]
    if has_next:
        in_specs.append(pl.BlockSpec((H, H), lambda i: (0, 0)))
        args.append(wea_next)
        out_specs = [pl.BlockSpec((tile, H), lambda i: (i, 0))] * 2
        out_shape = [jax.ShapeDtypeStruct((n, H), jnp.float32)] * 2
    else:
        out_specs = [pl.BlockSpec((tile, H), lambda i: (i, 0))]
        out_shape = [jax.ShapeDtypeStruct((n, H), jnp.float32)]
    res = pl.pallas_call(
        functools.partial(_node_body, has_next),
        grid=grid,
        in_specs=in_specs,
        out_specs=out_specs,
        out_shape=out_shape,
    )(*args)
    if has_next:
        return res[0], res[1]
    return res[0], None


def _rsqrt_scalar(x):
    xi = lax.bitcast_convert_type(x, jnp.int32)
    xi = jnp.int32(0x5F3759DF) - lax.shift_right_logical(xi, 1)
    y = lax.bitcast_convert_type(xi, jnp.float32)
    xh = x * 0.5
    for _ in range(3):
        y = y * (1.5 - xh * y * y)
    return y


def _proj_sc_body(ew_hbm, hw_hbm, src_hbm, dst_hbm, w_hbm, eproj_hbm,
                  ew_v, gs_v, gd_v, sidx_v, didx_v, w_v,
                  sem_idx, sem_ew, sem_g, sem_out):
    cid = lax.axis_index("c")
    sid = lax.axis_index("s")
    wid = cid * NS + sid
    base0 = wid * EPT

    def _in_descs(c, b):
        base = base0 + c * CHUNK
        return [
            (src_hbm.at[pl.ds(base, CHUNK)], sidx_v.at[b], sem_idx.at[b]),
            (dst_hbm.at[pl.ds(base, CHUNK)], didx_v.at[b], sem_idx.at[b]),
            (w_hbm.at[pl.ds(base, CHUNK)], w_v.at[b], sem_idx.at[b]),
        ]

    def _issue_in(c, b):
        base = base0 + c * CHUNK
        for s, d, sem in _in_descs(c, b):
            pltpu.async_copy(s, d, sem)
        pltpu.async_copy(ew_hbm.at[pl.ds(base, CHUNK), :], ew_v.at[b],
                         sem_ew.at[b])

    def _wait_idx(c, b):
        for s, d, sem in _in_descs(c, b):
            pltpu.make_async_copy(s, d, sem).wait()

    def _issue_gather(c, b):
        pltpu.async_copy(hw_hbm.at[sidx_v.at[b]], gs_v.at[b], sem_g.at[b])
        pltpu.async_copy(hw_hbm.at[didx_v.at[b]], gd_v.at[b], sem_g.at[b])

    def _wait_ew_g(c, b):
        base = base0 + c * CHUNK
        pltpu.make_async_copy(ew_hbm.at[pl.ds(base, CHUNK), :], ew_v.at[b],
                              sem_ew.at[b]).wait()
        pltpu.make_async_copy(hw_hbm.at[sidx_v.at[b]], gs_v.at[b],
                              sem_g.at[b]).wait()
        pltpu.make_async_copy(hw_hbm.at[didx_v.at[b]], gd_v.at[b],
                              sem_g.at[b]).wait()

    def _issue_out(c, b):
        base = base0 + c * CHUNK
        pltpu.async_copy(ew_v.at[b], eproj_hbm.at[pl.ds(base, CHUNK), :],
                         sem_out.at[b])

    def _wait_out(c, b):
        base = base0 + c * CHUNK
        pltpu.make_async_copy(ew_v.at[b], eproj_hbm.at[pl.ds(base, CHUNK), :],
                              sem_out.at[b]).wait()

    def _compute(c, b):
        def _group(g, _):
            wg = w_v[b, pl.ds(g * LANES, LANES)]
            for j in range(LANES):
                i = g * LANES + j
                us = []
                acc = None
                for k in range(KV):
                    sl = pl.ds(k * LANES, LANES)
                    u = ew_v[b, i, sl] + gs_v[b, i, sl] - gd_v[b, i, sl]
                    us.append(u)
                    acc = u * u if acc is None else acc + u * u
                terms = [acc[j2] for j2 in range(LANES)]
                while len(terms) > 1:
                    terms = [terms[i2] + terms[i2 + 1]
                             for i2 in range(0, len(terms), 2)]
                ss = jnp.maximum(terms[0], 1e-16)
                rs = _rsqrt_scalar(ss)
                scale_s = jnp.minimum(wg[j] * rs, 1.0)
                scale = jnp.full((LANES,), scale_s, jnp.float32)
                for k in range(KV):
                    ew_v[b, i, pl.ds(k * LANES, LANES)] = us[k] * scale
            return 0
        lax.fori_loop(0, CHUNK // LANES, _group, 0)

    _issue_in(0, 0)
    _issue_in(1, 1)
    _wait_idx(0, 0)
    _issue_gather(0, 0)

    def _step(c, b, static=False):
        b1 = (b + 1) % NBUF
        b2 = (b + 2) % NBUF

        def _in_part():
            _issue_in(c + 2, b2)

        def _inwait_part():
            _wait_out(c + 2 - NBUF, b2)

        def _g_part():
            _wait_idx(c + 1, b1)
            _issue_gather(c + 1, b1)

        if static:
            if c + 2 < NCHUNK:
                if c + 2 - NBUF >= 0:
                    _inwait_part()
                _in_part()
            if c + 1 < NCHUNK:
                _g_part()
        else:
            @pl.when(c + 2 < NCHUNK)
            def _():
                @pl.when(c + 2 - NBUF >= 0)
                def _():
                    _inwait_part()
                _in_part()

            @pl.when(c + 1 < NCHUNK)
            def _():
                _g_part()

        _wait_ew_g(c, b)
        _compute(c, b)
        _issue_out(c, b)

    def _groupn(g, _):
        for j in range(NBUF):
            _step(g * NBUF + j, j)
        return 0

    n_main = (NCHUNK // NBUF) * NBUF
    lax.fori_loop(0, NCHUNK // NBUF, _groupn, 0)
    for c in range(n_main, NCHUNK):
        _step(c, c % NBUF, static=True)
    for c in range(NCHUNK - NBUF, NCHUNK):
        _wait_out(c, c % NBUF)


def _proj_stage(ew, hw, src, dst, w):
    mesh = plsc.VectorSubcoreMesh(core_axis_name="c", subcore_axis_name="s",
                                  num_cores=NC, num_subcores=NS)
    scratch = [
        pltpu.VMEM((NBUF, CHUNK, H), jnp.float32),
        pltpu.VMEM((NBUF, CHUNK, H), jnp.float32),
        pltpu.VMEM((NBUF, CHUNK, H), jnp.float32),
        pltpu.VMEM((NBUF, CHUNK), jnp.int32),
        pltpu.VMEM((NBUF, CHUNK), jnp.int32),
        pltpu.VMEM((NBUF, CHUNK), jnp.float32),
        pltpu.SemaphoreType.DMA((NBUF,)),
        pltpu.SemaphoreType.DMA((NBUF,)),
        pltpu.SemaphoreType.DMA((NBUF,)),
        pltpu.SemaphoreType.DMA((NBUF,)),
    ]
    fn = pl.kernel(
        _proj_sc_body,
        out_type=jax.ShapeDtypeStruct((N_EDGES, H), jnp.float32),
        mesh=mesh,
        scratch_types=scratch,
    )
    return fn(ew, hw, src, dst, w)


def _agg_sc_body(with_deg, eproj_hbm, dst_hbm, zeros_hbm, zeros1_hbm,
                 agg_hbm, deg_hbm,
                 ep_v, didx_v, ones_v, agg_sp, deg_sp, sem_in, sem_out):
    cid = lax.axis_index("c")
    sid = lax.axis_index("s")
    wid = cid * NS + sid
    rows_pt = N_PAD // NS

    pltpu.sync_copy(zeros_hbm.at[pl.ds(sid * rows_pt, rows_pt), :],
                    agg_sp.at[pl.ds(sid * rows_pt, rows_pt), :])
    if with_deg:
        for g in range(CHUNK // LANES):
            ones_v[pl.ds(g * LANES, LANES)] = jnp.full((LANES,), 1.0,
                                                       jnp.float32)
        @pl.when(sid == 0)
        def _():
            pltpu.sync_copy(zeros1_hbm, deg_sp)
    plsc.subcore_barrier()

    base0 = wid * EPT

    def _in_descs(c, b):
        base = base0 + c * CHUNK
        return [
            (eproj_hbm.at[pl.ds(base, CHUNK), :], ep_v.at[b], sem_in.at[b]),
            (dst_hbm.at[pl.ds(base, CHUNK)], didx_v.at[b], sem_in.at[b]),
        ]

    def _issue_in(c, b):
        for s, d, sem in _in_descs(c, b):
            pltpu.async_copy(s, d, sem)

    def _wait_in(c, b):
        for s, d, sem in _in_descs(c, b):
            pltpu.make_async_copy(s, d, sem).wait()

    def _out_descs(c, b):
        descs = [(ep_v.at[b], agg_sp.at[didx_v.at[b]])]
        if with_deg:
            descs.append((ones_v, deg_sp.at[didx_v.at[b]]))
        return descs

    def _issue_out(c, b):
        for s, d in _out_descs(c, b):
            pltpu.async_copy(s, d, sem_out.at[b], add=True)

    def _wait_out(c, b):
        for s, d in _out_descs(c, b):
            pltpu.make_async_copy(s, d, sem_out.at[b]).wait()

    _issue_in(0, 0)
    _issue_in(1, 1)

    def _step(c, b, static=False):
        b2 = (b + 2) % NBUF

        def _in_part():
            _issue_in(c + 2, b2)

        def _inwait_part():
            _wait_out(c + 2 - NBUF, b2)

        if static:
            if c + 2 < NCHUNK:
                if c + 2 - NBUF >= 0:
                    _inwait_part()
                _in_part()
        else:
            @pl.when(c + 2 < NCHUNK)
            def _():
                @pl.when(c + 2 - NBUF >= 0)
                def _():
                    _inwait_part()
                _in_part()

        _wait_in(c, b)
        _issue_out(c, b)

    def _groupn(g, _):
        for j in range(NBUF):
            _step(g * NBUF + j, j)
        return 0

    n_main = (NCHUNK // NBUF) * NBUF
    lax.fori_loop(0, NCHUNK // NBUF, _groupn, 0)
    for c in range(n_main, NCHUNK):
        _step(c, c % NBUF, static=True)
    for c in range(NCHUNK - NBUF, NCHUNK):
        _wait_out(c, c % NBUF)

    plsc.subcore_barrier()

    pltpu.sync_copy(agg_sp.at[pl.ds(sid * rows_pt, rows_pt), :],
                    agg_hbm.at[cid, pl.ds(sid * rows_pt, rows_pt), :])
    if with_deg:
        pltpu.sync_copy(deg_sp.at[pl.ds(sid * rows_pt, rows_pt)],
                        deg_hbm.at[cid, pl.ds(sid * rows_pt, rows_pt)])


def _agg_stage(eproj, dst, zeros_big, zeros_1d, with_deg):
    mesh = plsc.VectorSubcoreMesh(core_axis_name="c", subcore_axis_name="s",
                                  num_cores=NC, num_subcores=NS)
    out_type = [
        jax.ShapeDtypeStruct((NC, N_PAD, H), jnp.float32),
        jax.ShapeDtypeStruct((NC, N_PAD), jnp.float32),
    ]
    scratch = [
        pltpu.VMEM((NBUF, CHUNK, H), jnp.float32),
        pltpu.VMEM((NBUF, CHUNK), jnp.int32),
        pltpu.VMEM((CHUNK,), jnp.float32),
        pltpu.VMEM_SHARED((N_PAD, H), jnp.float32),
        pltpu.VMEM_SHARED((N_PAD,), jnp.float32),
        pltpu.SemaphoreType.DMA((NBUF,)),
        pltpu.SemaphoreType.DMA((NBUF,)),
    ]
    fn = pl.kernel(
        functools.partial(_agg_sc_body, with_deg),
        out_type=out_type,
        mesh=mesh,
        scratch_types=scratch,
    )
    return fn(eproj, dst, zeros_big, zeros_1d)


def kernel(h, e, edge_index, w, params):
    src = edge_index[0].astype(jnp.int32)
    dst = edge_index[1].astype(jnp.int32)
    w = w.astype(jnp.float32)
    zeros_big = jnp.zeros((N_PAD, H), jnp.float32)
    zeros_1d = jnp.zeros((N_PAD,), jnp.float32)

    hw = _project(h, params[0]["W_ea"])
    h_cur, e_cur = h, e
    deg_parts = None
    for li, p in enumerate(params):
        bias = (p["b_eu"] + p["b_ea"]).astype(jnp.float32)
        ew = _linear(e_cur, p["W_eu"], bias, tile=5000)
        e_proj = _proj_stage(ew, hw, src, dst, w)
        agg_pad, deg_out = _agg_stage(e_proj, dst, zeros_big, zeros_1d,
                                      with_deg=(li == 0))
        agg_parts = agg_pad[:, :N_NODES, :]
        if li == 0:
            deg_parts = deg_out[:, :N_NODES].reshape(NC, N_NODES, 1)
        w1 = p["W_n1"]
        w1h = w1[:, :H]
        w1a = w1[:, H:]
        wea_next = params[li + 1]["W_ea"] if li + 1 < len(params) else None
        h_cur, hw = _node_update(
            h_cur, agg_parts, deg_parts, w1h, w1a, p["b_n1"], p["W_n2"],
            p["b_n2"], wea_next)
        e_cur = e_proj
    return (h_cur, e_cur)


# speedup vs baseline: 1.3516x; 1.3516x over previous
import functools

import jax
import jax.numpy as jnp
from jax import lax
from jax.experimental import pallas as pl
from jax.experimental.pallas import tpu as pltpu
from jax.experimental.pallas import tpu_sc as plsc

N_NODES = 10000
N_EDGES = 320000
H = 128

NC = 2
NS = 16
NW = NC * NS
EPT = N_EDGES // NW
CHUNK = 80
NCHUNK = EPT // CHUNK
LANES = 16
KV = H // LANES
NBUF = 3

N_PAD = 10240


def _linear_body(x_ref, w_ref, b_ref, o_ref):
    x = x_ref[...]
    w = w_ref[...]
    y = lax.dot_general(x, w, (((1,), (1,)), ((), ())),
                        preferred_element_type=jnp.float32)
    o_ref[...] = y + b_ref[...]


def _linear(x, w, b, tile):
    m, k = x.shape
    out = w.shape[0]
    grid = (m // tile,)
    return pl.pallas_call(
        _linear_body,
        grid=grid,
        in_specs=[
            pl.BlockSpec((tile, k), lambda i: (i, 0)),
            pl.BlockSpec((out, k), lambda i: (0, 0)),
            pl.BlockSpec((1, out), lambda i: (0, 0)),
        ],
        out_specs=pl.BlockSpec((tile, out), lambda i: (i, 0)),
        out_shape=jax.ShapeDtypeStruct((m, out), jnp.float32),
    )(x, w, b.reshape(1, out))


def _project_body(x_ref, w_ref, hw_ref):
    hw_ref[...] = lax.dot_general(
        x_ref[...], w_ref[...], (((1,), (1,)), ((), ())),
        preferred_element_type=jnp.float32)


def _project(x, w, tile=2000):
    m, k = x.shape
    out = w.shape[0]
    grid = (m // tile,)
    return pl.pallas_call(
        _project_body,
        grid=grid,
        in_specs=[
            pl.BlockSpec((tile, k), lambda i: (i, 0)),
            pl.BlockSpec((out, k), lambda i: (0, 0)),
        ],
        out_specs=pl.BlockSpec((tile, out), lambda i: (i, 0)),
        out_shape=jax.ShapeDtypeStruct((m, out), jnp.float32),
    )(x, w)


def _node_body(has_next, h_ref, ap_ref, dp_ref, w1h_ref, w1a_ref, b1_ref,
               w2_ref, b2_ref, *rest):
    if has_next:
        wea_ref, hn_ref, hw_ref = rest
    else:
        (hn_ref,) = rest
    deg = dp_ref[0] + dp_ref[1]
    rdeg = 1.0 / jnp.maximum(deg, 1.0)
    agg = (ap_ref[0] + ap_ref[1]) * rdeg
    x1 = lax.dot_general(h_ref[...], w1h_ref[...], (((1,), (1,)), ((), ())),
                         preferred_element_type=jnp.float32)
    x1 = x1 + lax.dot_general(agg, w1a_ref[...], (((1,), (1,)), ((), ())),
                              preferred_element_type=jnp.float32)
    x1 = x1 + b1_ref[...]
    hid = x1 * jax.nn.sigmoid(x1)
    h_new = lax.dot_general(hid, w2_ref[...], (((1,), (1,)), ((), ())),
                            preferred_element_type=jnp.float32)
    h_new = h_new + b2_ref[...]
    hn_ref[...] = h_new
    if has_next:
        hw_ref[...] = lax.dot_general(
            h_new, wea_ref[...], (((1,), (1,)), ((), ())),
            preferred_element_type=jnp.float32)


def _node_update(h, agg_parts, deg_parts, w1h, w1a, b1, w2, b2, wea_next,
                 tile=2000):
    n = h.shape[0]
    grid = (n // tile,)
    has_next = wea_next is not None
    in_specs = [
        pl.BlockSpec((tile, H), lambda i: (i, 0)),
        pl.BlockSpec((2, tile, H), lambda i: (0, i, 0)),
        pl.BlockSpec((2, tile, 1), lambda i: (0, i, 0)),
        pl.BlockSpec((H, H), lambda i: (0, 0)),
        pl.BlockSpec((H, H), lambda i: (0, 0)),
        pl.BlockSpec((1, H), lambda i: (0, 0)),
        pl.BlockSpec((H, H), lambda i: (0, 0)),
        pl.BlockSpec((1, H), lambda i: (0, 0)),
    ]
    args = [h, agg_parts, deg_parts, w1h, w1a, b1.reshape(1, H), w2,
            b2.reshape(1, H)]
    if has_next:
        in_specs.append(pl.BlockSpec((H, H), lambda i: (0, 0)))
        args.append(wea_next)
        out_specs = [pl.BlockSpec((tile, H), lambda i: (i, 0))] * 2
        out_shape = [jax.ShapeDtypeStruct((n, H), jnp.float32)] * 2
    else:
        out_specs = [pl.BlockSpec((tile, H), lambda i: (i, 0))]
        out_shape = [jax.ShapeDtypeStruct((n, H), jnp.float32)]
    res = pl.pallas_call(
        functools.partial(_node_body, has_next),
        grid=grid,
        in_specs=in_specs,
        out_specs=out_specs,
        out_shape=out_shape,
    )(*args)
    if has_next:
        return res[0], res[1]
    return res[0], None


def _hsum16(v):
    dnums = lax.GatherDimensionNumbers(
        offset_dims=(), collapsed_slice_dims=(0,), start_index_map=(0,))
    for k in (8, 4, 2, 1):
        perm = lax.iota(jnp.int32, LANES) ^ k
        v = v + lax.gather(v, perm[:, None], dimension_numbers=dnums,
                           slice_sizes=(1,),
                           mode=lax.GatherScatterMode.PROMISE_IN_BOUNDS)
    return v


def _rsqrt_scalar(x):
    xi = lax.bitcast_convert_type(x, jnp.int32)
    xi = jnp.int32(0x5F3759DF) - lax.shift_right_logical(xi, 1)
    y = lax.bitcast_convert_type(xi, jnp.float32)
    xh = x * 0.5
    for _ in range(3):
        y = y * (1.5 - xh * y * y)
    return y


def _proj_sc_body(ew_hbm, hw_hbm, src_hbm, dst_hbm, w_hbm, eproj_hbm,
                  ew_v, gs_v, gd_v, sidx_v, didx_v, w_v,
                  sem_idx, sem_ew, sem_g, sem_out):
    cid = lax.axis_index("c")
    sid = lax.axis_index("s")
    wid = cid * NS + sid
    base0 = wid * EPT

    def _in_descs(c, b):
        base = base0 + c * CHUNK
        return [
            (src_hbm.at[pl.ds(base, CHUNK)], sidx_v.at[b], sem_idx.at[b]),
            (dst_hbm.at[pl.ds(base, CHUNK)], didx_v.at[b], sem_idx.at[b]),
            (w_hbm.at[pl.ds(base, CHUNK)], w_v.at[b], sem_idx.at[b]),
        ]

    def _issue_in(c, b):
        base = base0 + c * CHUNK
        for s, d, sem in _in_descs(c, b):
            pltpu.async_copy(s, d, sem)
        pltpu.async_copy(ew_hbm.at[pl.ds(base, CHUNK), :], ew_v.at[b],
                         sem_ew.at[b])

    def _wait_idx(c, b):
        for s, d, sem in _in_descs(c, b):
            pltpu.make_async_copy(s, d, sem).wait()

    def _issue_gather(c, b):
        pltpu.async_copy(hw_hbm.at[sidx_v.at[b]], gs_v.at[b], sem_g.at[b])
        pltpu.async_copy(hw_hbm.at[didx_v.at[b]], gd_v.at[b], sem_g.at[b])

    def _wait_ew_g(c, b):
        base = base0 + c * CHUNK
        pltpu.make_async_copy(ew_hbm.at[pl.ds(base, CHUNK), :], ew_v.at[b],
                              sem_ew.at[b]).wait()
        pltpu.make_async_copy(hw_hbm.at[sidx_v.at[b]], gs_v.at[b],
                              sem_g.at[b]).wait()
        pltpu.make_async_copy(hw_hbm.at[didx_v.at[b]], gd_v.at[b],
                              sem_g.at[b]).wait()

    def _issue_out(c, b):
        base = base0 + c * CHUNK
        pltpu.async_copy(ew_v.at[b], eproj_hbm.at[pl.ds(base, CHUNK), :],
                         sem_out.at[b])

    def _wait_out(c, b):
        base = base0 + c * CHUNK
        pltpu.make_async_copy(ew_v.at[b], eproj_hbm.at[pl.ds(base, CHUNK), :],
                              sem_out.at[b]).wait()

    def _compute(c, b):
        @plsc.parallel_loop(0, CHUNK // LANES)
        def _group(g):
            wg = w_v[b, pl.ds(g * LANES, LANES)]
            for j in range(LANES):
                i = g * LANES + j
                us = []
                acc = None
                for k in range(KV):
                    sl = pl.ds(k * LANES, LANES)
                    u = ew_v[b, i, sl] + gs_v[b, i, sl] - gd_v[b, i, sl]
                    us.append(u)
                    acc = u * u if acc is None else acc + u * u
                ssv = _hsum16(acc)
                ss = jnp.maximum(ssv[0], 1e-16)
                rs = _rsqrt_scalar(ss)
                scale_s = jnp.minimum(wg[j] * rs, 1.0)
                scale = jnp.full((LANES,), scale_s, jnp.float32)
                for k in range(KV):
                    ew_v[b, i, pl.ds(k * LANES, LANES)] = us[k] * scale

    _issue_in(0, 0)
    _issue_in(1, 1)
    _wait_idx(0, 0)
    _issue_gather(0, 0)

    def _step(c, b, static=False):
        b1 = (b + 1) % NBUF
        b2 = (b + 2) % NBUF

        def _in_part():
            _issue_in(c + 2, b2)

        def _inwait_part():
            _wait_out(c + 2 - NBUF, b2)

        def _g_part():
            _wait_idx(c + 1, b1)
            _issue_gather(c + 1, b1)

        if static:
            if c + 2 < NCHUNK:
                if c + 2 - NBUF >= 0:
                    _inwait_part()
                _in_part()
            if c + 1 < NCHUNK:
                _g_part()
        else:
            @pl.when(c + 2 < NCHUNK)
            def _():
                @pl.when(c + 2 - NBUF >= 0)
                def _():
                    _inwait_part()
                _in_part()

            @pl.when(c + 1 < NCHUNK)
            def _():
                _g_part()

        _wait_ew_g(c, b)
        _compute(c, b)
        _issue_out(c, b)

    def _groupn(g, _):
        for j in range(NBUF):
            _step(g * NBUF + j, j)
        return 0

    n_main = (NCHUNK // NBUF) * NBUF
    lax.fori_loop(0, NCHUNK // NBUF, _groupn, 0)
    for c in range(n_main, NCHUNK):
        _step(c, c % NBUF, static=True)
    for c in range(NCHUNK - NBUF, NCHUNK):
        _wait_out(c, c % NBUF)


def _proj_stage(ew, hw, src, dst, w):
    mesh = plsc.VectorSubcoreMesh(core_axis_name="c", subcore_axis_name="s",
                                  num_cores=NC, num_subcores=NS)
    scratch = [
        pltpu.VMEM((NBUF, CHUNK, H), jnp.float32),
        pltpu.VMEM((NBUF, CHUNK, H), jnp.float32),
        pltpu.VMEM((NBUF, CHUNK, H), jnp.float32),
        pltpu.VMEM((NBUF, CHUNK), jnp.int32),
        pltpu.VMEM((NBUF, CHUNK), jnp.int32),
        pltpu.VMEM((NBUF, CHUNK), jnp.float32),
        pltpu.SemaphoreType.DMA((NBUF,)),
        pltpu.SemaphoreType.DMA((NBUF,)),
        pltpu.SemaphoreType.DMA((NBUF,)),
        pltpu.SemaphoreType.DMA((NBUF,)),
    ]
    fn = pl.kernel(
        _proj_sc_body,
        out_type=jax.ShapeDtypeStruct((N_EDGES, H), jnp.float32),
        mesh=mesh,
        scratch_types=scratch,
    )
    return fn(ew, hw, src, dst, w)


def _agg_sc_body(with_deg, eproj_hbm, dst_hbm, zeros_hbm, zeros1_hbm,
                 agg_hbm, deg_hbm,
                 ep_v, didx_v, ones_v, agg_sp, deg_sp, sem_in, sem_out):
    cid = lax.axis_index("c")
    sid = lax.axis_index("s")
    wid = cid * NS + sid
    rows_pt = N_PAD // NS

    pltpu.sync_copy(zeros_hbm.at[pl.ds(sid * rows_pt, rows_pt), :],
                    agg_sp.at[pl.ds(sid * rows_pt, rows_pt), :])
    if with_deg:
        for g in range(CHUNK // LANES):
            ones_v[pl.ds(g * LANES, LANES)] = jnp.full((LANES,), 1.0,
                                                       jnp.float32)
        @pl.when(sid == 0)
        def _():
            pltpu.sync_copy(zeros1_hbm, deg_sp)
    plsc.subcore_barrier()

    base0 = wid * EPT

    def _in_descs(c, b):
        base = base0 + c * CHUNK
        return [
            (eproj_hbm.at[pl.ds(base, CHUNK), :], ep_v.at[b], sem_in.at[b]),
            (dst_hbm.at[pl.ds(base, CHUNK)], didx_v.at[b], sem_in.at[b]),
        ]

    def _issue_in(c, b):
        for s, d, sem in _in_descs(c, b):
            pltpu.async_copy(s, d, sem)

    def _wait_in(c, b):
        for s, d, sem in _in_descs(c, b):
            pltpu.make_async_copy(s, d, sem).wait()

    def _out_descs(c, b):
        descs = [(ep_v.at[b], agg_sp.at[didx_v.at[b]])]
        if with_deg:
            descs.append((ones_v, deg_sp.at[didx_v.at[b]]))
        return descs

    def _issue_out(c, b):
        for s, d in _out_descs(c, b):
            pltpu.async_copy(s, d, sem_out.at[b], add=True)

    def _wait_out(c, b):
        for s, d in _out_descs(c, b):
            pltpu.make_async_copy(s, d, sem_out.at[b]).wait()

    _issue_in(0, 0)
    _issue_in(1, 1)

    def _step(c, b, static=False):
        b2 = (b + 2) % NBUF

        def _in_part():
            _issue_in(c + 2, b2)

        def _inwait_part():
            _wait_out(c + 2 - NBUF, b2)

        if static:
            if c + 2 < NCHUNK:
                if c + 2 - NBUF >= 0:
                    _inwait_part()
                _in_part()
        else:
            @pl.when(c + 2 < NCHUNK)
            def _():
                @pl.when(c + 2 - NBUF >= 0)
                def _():
                    _inwait_part()
                _in_part()

        _wait_in(c, b)
        _issue_out(c, b)

    def _groupn(g, _):
        for j in range(NBUF):
            _step(g * NBUF + j, j)
        return 0

    n_main = (NCHUNK // NBUF) * NBUF
    lax.fori_loop(0, NCHUNK // NBUF, _groupn, 0)
    for c in range(n_main, NCHUNK):
        _step(c, c % NBUF, static=True)
    for c in range(NCHUNK - NBUF, NCHUNK):
        _wait_out(c, c % NBUF)

    plsc.subcore_barrier()

    pltpu.sync_copy(agg_sp.at[pl.ds(sid * rows_pt, rows_pt), :],
                    agg_hbm.at[cid, pl.ds(sid * rows_pt, rows_pt), :])
    if with_deg:
        pltpu.sync_copy(deg_sp.at[pl.ds(sid * rows_pt, rows_pt)],
                        deg_hbm.at[cid, pl.ds(sid * rows_pt, rows_pt)])


def _agg_stage(eproj, dst, zeros_big, zeros_1d, with_deg):
    mesh = plsc.VectorSubcoreMesh(core_axis_name="c", subcore_axis_name="s",
                                  num_cores=NC, num_subcores=NS)
    out_type = [
        jax.ShapeDtypeStruct((NC, N_PAD, H), jnp.float32),
        jax.ShapeDtypeStruct((NC, N_PAD), jnp.float32),
    ]
    scratch = [
        pltpu.VMEM((NBUF, CHUNK, H), jnp.float32),
        pltpu.VMEM((NBUF, CHUNK), jnp.int32),
        pltpu.VMEM((CHUNK,), jnp.float32),
        pltpu.VMEM_SHARED((N_PAD, H), jnp.float32),
        pltpu.VMEM_SHARED((N_PAD,), jnp.float32),
        pltpu.SemaphoreType.DMA((NBUF,)),
        pltpu.SemaphoreType.DMA((NBUF,)),
    ]
    fn = pl.kernel(
        functools.partial(_agg_sc_body, with_deg),
        out_type=out_type,
        mesh=mesh,
        scratch_types=scratch,
    )
    return fn(eproj, dst, zeros_big, zeros_1d)


def kernel(h, e, edge_index, w, params):
    src = edge_index[0].astype(jnp.int32)
    dst = edge_index[1].astype(jnp.int32)
    w = w.astype(jnp.float32)
    zeros_big = jnp.zeros((N_PAD, H), jnp.float32)
    zeros_1d = jnp.zeros((N_PAD,), jnp.float32)

    hw = _project(h, params[0]["W_ea"])
    h_cur, e_cur = h, e
    deg_parts = None
    for li, p in enumerate(params):
        bias = (p["b_eu"] + p["b_ea"]).astype(jnp.float32)
        ew = _linear(e_cur, p["W_eu"], bias, tile=5000)
        e_proj = _proj_stage(ew, hw, src, dst, w)
        agg_pad, deg_out = _agg_stage(e_proj, dst, zeros_big, zeros_1d,
                                      with_deg=(li == 0))
        agg_parts = agg_pad[:, :N_NODES, :]
        if li == 0:
            deg_parts = deg_out[:, :N_NODES].reshape(NC, N_NODES, 1)
        w1 = p["W_n1"]
        w1h = w1[:, :H]
        w1a = w1[:, H:]
        wea_next = params[li + 1]["W_ea"] if li + 1 < len(params) else None
        h_cur, hw = _node_update(
            h_cur, agg_parts, deg_parts, w1h, w1a, p["b_n1"], p["W_n2"],
            p["b_n2"], wea_next)
        e_cur = e_proj
    return (h_cur, e_cur)


# speedup vs baseline: 1.4071x; 1.0410x over previous
import functools

import jax
import jax.numpy as jnp
from jax import lax
from jax.experimental import pallas as pl
from jax.experimental.pallas import tpu as pltpu
from jax.experimental.pallas import tpu_sc as plsc

N_NODES = 10000
N_EDGES = 320000
H = 128

NC = 2
NS = 16
NW = NC * NS
EPT = N_EDGES // NW
CHUNK = 80
NCHUNK = EPT // CHUNK
LANES = 16
KV = H // LANES
NBUF = 3

N_PAD = 10240


def _linear_body(x_ref, w_ref, b_ref, o_ref):
    x = x_ref[...]
    w = w_ref[...]
    y = lax.dot_general(x, w, (((1,), (1,)), ((), ())),
                        preferred_element_type=jnp.float32)
    o_ref[...] = y + b_ref[...]


def _linear(x, w, b, tile):
    m, k = x.shape
    out = w.shape[0]
    grid = (m // tile,)
    return pl.pallas_call(
        _linear_body,
        grid=grid,
        in_specs=[
            pl.BlockSpec((tile, k), lambda i: (i, 0)),
            pl.BlockSpec((out, k), lambda i: (0, 0)),
            pl.BlockSpec((1, out), lambda i: (0, 0)),
        ],
        out_specs=pl.BlockSpec((tile, out), lambda i: (i, 0)),
        out_shape=jax.ShapeDtypeStruct((m, out), jnp.float32),
    )(x, w, b.reshape(1, out))


def _project_body(x_ref, w_ref, hw_ref):
    hw_ref[...] = lax.dot_general(
        x_ref[...], w_ref[...], (((1,), (1,)), ((), ())),
        preferred_element_type=jnp.float32)


def _project(x, w, tile=2000):
    m, k = x.shape
    out = w.shape[0]
    grid = (m // tile,)
    return pl.pallas_call(
        _project_body,
        grid=grid,
        in_specs=[
            pl.BlockSpec((tile, k), lambda i: (i, 0)),
            pl.BlockSpec((out, k), lambda i: (0, 0)),
        ],
        out_specs=pl.BlockSpec((tile, out), lambda i: (i, 0)),
        out_shape=jax.ShapeDtypeStruct((m, out), jnp.float32),
    )(x, w)


def _node_body(has_next, h_ref, ap_ref, dp_ref, w1h_ref, w1a_ref, b1_ref,
               w2_ref, b2_ref, *rest):
    if has_next:
        wea_ref, hn_ref, hw_ref = rest
    else:
        (hn_ref,) = rest
    deg = dp_ref[0] + dp_ref[1]
    rdeg = 1.0 / jnp.maximum(deg, 1.0)
    agg = (ap_ref[0] + ap_ref[1]) * rdeg
    x1 = lax.dot_general(h_ref[...], w1h_ref[...], (((1,), (1,)), ((), ())),
                         preferred_element_type=jnp.float32)
    x1 = x1 + lax.dot_general(agg, w1a_ref[...], (((1,), (1,)), ((), ())),
                              preferred_element_type=jnp.float32)
    x1 = x1 + b1_ref[...]
    hid = x1 * jax.nn.sigmoid(x1)
    h_new = lax.dot_general(hid, w2_ref[...], (((1,), (1,)), ((), ())),
                            preferred_element_type=jnp.float32)
    h_new = h_new + b2_ref[...]
    hn_ref[...] = h_new
    if has_next:
        hw_ref[...] = lax.dot_general(
            h_new, wea_ref[...], (((1,), (1,)), ((), ())),
            preferred_element_type=jnp.float32)


def _node_update(h, agg_parts, deg_parts, w1h, w1a, b1, w2, b2, wea_next,
                 tile=2000):
    n = h.shape[0]
    grid = (n // tile,)
    has_next = wea_next is not None
    in_specs = [
        pl.BlockSpec((tile, H), lambda i: (i, 0)),
        pl.BlockSpec((2, tile, H), lambda i: (0, i, 0)),
        pl.BlockSpec((2, tile, 1), lambda i: (0, i, 0)),
        pl.BlockSpec((H, H), lambda i: (0, 0)),
        pl.BlockSpec((H, H), lambda i: (0, 0)),
        pl.BlockSpec((1, H), lambda i: (0, 0)),
        pl.BlockSpec((H, H), lambda i: (0, 0)),
        pl.BlockSpec((1, H), lambda i: (0, 0)),
    ]
    args = [h, agg_parts, deg_parts, w1h, w1a, b1.reshape(1, H), w2,
            b2.reshape(1, H)]
    if has_next:
        in_specs.append(pl.BlockSpec((H, H), lambda i: (0, 0)))
        args.append(wea_next)
        out_specs = [pl.BlockSpec((tile, H), lambda i: (i, 0))] * 2
        out_shape = [jax.ShapeDtypeStruct((n, H), jnp.float32)] * 2
    else:
        out_specs = [pl.BlockSpec((tile, H), lambda i: (i, 0))]
        out_shape = [jax.ShapeDtypeStruct((n, H), jnp.float32)]
    res = pl.pallas_call(
        functools.partial(_node_body, has_next),
        grid=grid,
        in_specs=in_specs,
        out_specs=out_specs,
        out_shape=out_shape,
    )(*args)
    if has_next:
        return res[0], res[1]
    return res[0], None


def _hsum16(v):
    dnums = lax.GatherDimensionNumbers(
        offset_dims=(), collapsed_slice_dims=(0,), start_index_map=(0,))
    for k in (8, 4, 2, 1):
        perm = lax.iota(jnp.int32, LANES) ^ k
        v = v + lax.gather(v, perm[:, None], dimension_numbers=dnums,
                           slice_sizes=(1,),
                           mode=lax.GatherScatterMode.PROMISE_IN_BOUNDS)
    return v


def _rsqrt_scalar(x):
    xi = lax.bitcast_convert_type(x, jnp.int32)
    xi = jnp.int32(0x5F3759DF) - lax.shift_right_logical(xi, 1)
    y = lax.bitcast_convert_type(xi, jnp.float32)
    xh = x * 0.5
    for _ in range(2):
        y = y * (1.5 - xh * y * y)
    return y


def _proj_sc_body(ew_hbm, hw_hbm, src_hbm, dst_hbm, w_hbm, eproj_hbm,
                  ew_v, gs_v, gd_v, sidx_v, didx_v, w_v,
                  sem_idx, sem_ew, sem_g, sem_out):
    cid = lax.axis_index("c")
    sid = lax.axis_index("s")
    wid = cid * NS + sid
    base0 = wid * EPT

    def _in_descs(c, b):
        base = base0 + c * CHUNK
        return [
            (src_hbm.at[pl.ds(base, CHUNK)], sidx_v.at[b], sem_idx.at[b]),
            (dst_hbm.at[pl.ds(base, CHUNK)], didx_v.at[b], sem_idx.at[b]),
            (w_hbm.at[pl.ds(base, CHUNK)], w_v.at[b], sem_idx.at[b]),
        ]

    def _issue_in(c, b):
        base = base0 + c * CHUNK
        for s, d, sem in _in_descs(c, b):
            pltpu.async_copy(s, d, sem)
        pltpu.async_copy(ew_hbm.at[pl.ds(base, CHUNK), :], ew_v.at[b],
                         sem_ew.at[b])

    def _wait_idx(c, b):
        for s, d, sem in _in_descs(c, b):
            pltpu.make_async_copy(s, d, sem).wait()

    def _issue_gather(c, b):
        pltpu.async_copy(hw_hbm.at[sidx_v.at[b]], gs_v.at[b], sem_g.at[b])
        pltpu.async_copy(hw_hbm.at[didx_v.at[b]], gd_v.at[b], sem_g.at[b])

    def _wait_ew_g(c, b):
        base = base0 + c * CHUNK
        pltpu.make_async_copy(ew_hbm.at[pl.ds(base, CHUNK), :], ew_v.at[b],
                              sem_ew.at[b]).wait()
        pltpu.make_async_copy(hw_hbm.at[sidx_v.at[b]], gs_v.at[b],
                              sem_g.at[b]).wait()
        pltpu.make_async_copy(hw_hbm.at[didx_v.at[b]], gd_v.at[b],
                              sem_g.at[b]).wait()

    def _issue_out(c, b):
        base = base0 + c * CHUNK
        pltpu.async_copy(ew_v.at[b], eproj_hbm.at[pl.ds(base, CHUNK), :],
                         sem_out.at[b])

    def _wait_out(c, b):
        base = base0 + c * CHUNK
        pltpu.make_async_copy(ew_v.at[b], eproj_hbm.at[pl.ds(base, CHUNK), :],
                              sem_out.at[b]).wait()

    def _compute(c, b):
        @plsc.parallel_loop(0, CHUNK // LANES)
        def _group(g):
            wg = w_v[b, pl.ds(g * LANES, LANES)]
            for j in range(LANES):
                i = g * LANES + j
                us = []
                acc = None
                for k in range(KV):
                    sl = pl.ds(k * LANES, LANES)
                    u = ew_v[b, i, sl] + gs_v[b, i, sl] - gd_v[b, i, sl]
                    us.append(u)
                    acc = u * u if acc is None else acc + u * u
                ssv = _hsum16(acc)
                ss = jnp.maximum(ssv[0], 1e-16)
                rs = _rsqrt_scalar(ss)
                scale_s = jnp.minimum(wg[j] * rs, 1.0)
                scale = jnp.full((LANES,), scale_s, jnp.float32)
                for k in range(KV):
                    ew_v[b, i, pl.ds(k * LANES, LANES)] = us[k] * scale

    _issue_in(0, 0)
    _issue_in(1, 1)
    _wait_idx(0, 0)
    _issue_gather(0, 0)

    def _step(c, b, static=False):
        b1 = (b + 1) % NBUF
        b2 = (b + 2) % NBUF

        def _in_part():
            _issue_in(c + 2, b2)

        def _inwait_part():
            _wait_out(c + 2 - NBUF, b2)

        def _g_part():
            _wait_idx(c + 1, b1)
            _issue_gather(c + 1, b1)

        if static:
            if c + 2 < NCHUNK:
                if c + 2 - NBUF >= 0:
                    _inwait_part()
                _in_part()
            if c + 1 < NCHUNK:
                _g_part()
        else:
            @pl.when(c + 2 < NCHUNK)
            def _():
                @pl.when(c + 2 - NBUF >= 0)
                def _():
                    _inwait_part()
                _in_part()

            @pl.when(c + 1 < NCHUNK)
            def _():
                _g_part()

        _wait_ew_g(c, b)
        _compute(c, b)
        _issue_out(c, b)

    def _groupn(g, _):
        for j in range(NBUF):
            _step(g * NBUF + j, j)
        return 0

    n_main = (NCHUNK // NBUF) * NBUF
    lax.fori_loop(0, NCHUNK // NBUF, _groupn, 0)
    for c in range(n_main, NCHUNK):
        _step(c, c % NBUF, static=True)
    for c in range(NCHUNK - NBUF, NCHUNK):
        _wait_out(c, c % NBUF)


def _proj_stage(ew, hw, src, dst, w):
    mesh = plsc.VectorSubcoreMesh(core_axis_name="c", subcore_axis_name="s",
                                  num_cores=NC, num_subcores=NS)
    scratch = [
        pltpu.VMEM((NBUF, CHUNK, H), jnp.float32),
        pltpu.VMEM((NBUF, CHUNK, H), jnp.float32),
        pltpu.VMEM((NBUF, CHUNK, H), jnp.float32),
        pltpu.VMEM((NBUF, CHUNK), jnp.int32),
        pltpu.VMEM((NBUF, CHUNK), jnp.int32),
        pltpu.VMEM((NBUF, CHUNK), jnp.float32),
        pltpu.SemaphoreType.DMA((NBUF,)),
        pltpu.SemaphoreType.DMA((NBUF,)),
        pltpu.SemaphoreType.DMA((NBUF,)),
        pltpu.SemaphoreType.DMA((NBUF,)),
    ]
    fn = pl.kernel(
        _proj_sc_body,
        out_type=jax.ShapeDtypeStruct((N_EDGES, H), jnp.float32),
        mesh=mesh,
        scratch_types=scratch,
    )
    return fn(ew, hw, src, dst, w)


def _agg_sc_body(with_deg, eproj_hbm, dst_hbm, zeros_hbm, zeros1_hbm,
                 agg_hbm, deg_hbm,
                 ep_v, didx_v, ones_v, agg_sp, deg_sp, sem_in, sem_out):
    cid = lax.axis_index("c")
    sid = lax.axis_index("s")
    wid = cid * NS + sid
    rows_pt = N_PAD // NS

    pltpu.sync_copy(zeros_hbm.at[pl.ds(sid * rows_pt, rows_pt), :],
                    agg_sp.at[pl.ds(sid * rows_pt, rows_pt), :])
    if with_deg:
        for g in range(CHUNK // LANES):
            ones_v[pl.ds(g * LANES, LANES)] = jnp.full((LANES,), 1.0,
                                                       jnp.float32)
        @pl.when(sid == 0)
        def _():
            pltpu.sync_copy(zeros1_hbm, deg_sp)
    plsc.subcore_barrier()

    base0 = wid * EPT

    def _in_descs(c, b):
        base = base0 + c * CHUNK
        return [
            (eproj_hbm.at[pl.ds(base, CHUNK), :], ep_v.at[b], sem_in.at[b]),
            (dst_hbm.at[pl.ds(base, CHUNK)], didx_v.at[b], sem_in.at[b]),
        ]

    def _issue_in(c, b):
        for s, d, sem in _in_descs(c, b):
            pltpu.async_copy(s, d, sem)

    def _wait_in(c, b):
        for s, d, sem in _in_descs(c, b):
            pltpu.make_async_copy(s, d, sem).wait()

    def _out_descs(c, b):
        descs = [(ep_v.at[b], agg_sp.at[didx_v.at[b]])]
        if with_deg:
            descs.append((ones_v, deg_sp.at[didx_v.at[b]]))
        return descs

    def _issue_out(c, b):
        for s, d in _out_descs(c, b):
            pltpu.async_copy(s, d, sem_out.at[b], add=True)

    def _wait_out(c, b):
        for s, d in _out_descs(c, b):
            pltpu.make_async_copy(s, d, sem_out.at[b]).wait()

    _issue_in(0, 0)
    _issue_in(1, 1)

    def _step(c, b, static=False):
        b2 = (b + 2) % NBUF

        def _in_part():
            _issue_in(c + 2, b2)

        def _inwait_part():
            _wait_out(c + 2 - NBUF, b2)

        if static:
            if c + 2 < NCHUNK:
                if c + 2 - NBUF >= 0:
                    _inwait_part()
                _in_part()
        else:
            @pl.when(c + 2 < NCHUNK)
            def _():
                @pl.when(c + 2 - NBUF >= 0)
                def _():
                    _inwait_part()
                _in_part()

        _wait_in(c, b)
        _issue_out(c, b)

    def _groupn(g, _):
        for j in range(NBUF):
            _step(g * NBUF + j, j)
        return 0

    n_main = (NCHUNK // NBUF) * NBUF
    lax.fori_loop(0, NCHUNK // NBUF, _groupn, 0)
    for c in range(n_main, NCHUNK):
        _step(c, c % NBUF, static=True)
    for c in range(NCHUNK - NBUF, NCHUNK):
        _wait_out(c, c % NBUF)

    plsc.subcore_barrier()

    pltpu.sync_copy(agg_sp.at[pl.ds(sid * rows_pt, rows_pt), :],
                    agg_hbm.at[cid, pl.ds(sid * rows_pt, rows_pt), :])
    if with_deg:
        pltpu.sync_copy(deg_sp.at[pl.ds(sid * rows_pt, rows_pt)],
                        deg_hbm.at[cid, pl.ds(sid * rows_pt, rows_pt)])


def _agg_stage(eproj, dst, zeros_big, zeros_1d, with_deg):
    mesh = plsc.VectorSubcoreMesh(core_axis_name="c", subcore_axis_name="s",
                                  num_cores=NC, num_subcores=NS)
    out_type = [
        jax.ShapeDtypeStruct((NC, N_PAD, H), jnp.float32),
        jax.ShapeDtypeStruct((NC, N_PAD), jnp.float32),
    ]
    scratch = [
        pltpu.VMEM((NBUF, CHUNK, H), jnp.float32),
        pltpu.VMEM((NBUF, CHUNK), jnp.int32),
        pltpu.VMEM((CHUNK,), jnp.float32),
        pltpu.VMEM_SHARED((N_PAD, H), jnp.float32),
        pltpu.VMEM_SHARED((N_PAD,), jnp.float32),
        pltpu.SemaphoreType.DMA((NBUF,)),
        pltpu.SemaphoreType.DMA((NBUF,)),
    ]
    fn = pl.kernel(
        functools.partial(_agg_sc_body, with_deg),
        out_type=out_type,
        mesh=mesh,
        scratch_types=scratch,
    )
    return fn(eproj, dst, zeros_big, zeros_1d)


def kernel(h, e, edge_index, w, params):
    src = edge_index[0].astype(jnp.int32)
    dst = edge_index[1].astype(jnp.int32)
    w = w.astype(jnp.float32)
    zeros_big = jnp.zeros((N_PAD, H), jnp.float32)
    zeros_1d = jnp.zeros((N_PAD,), jnp.float32)

    hw = _project(h, params[0]["W_ea"])
    h_cur, e_cur = h, e
    deg_parts = None
    for li, p in enumerate(params):
        bias = (p["b_eu"] + p["b_ea"]).astype(jnp.float32)
        ew = _linear(e_cur, p["W_eu"], bias, tile=5000)
        e_proj = _proj_stage(ew, hw, src, dst, w)
        agg_pad, deg_out = _agg_stage(e_proj, dst, zeros_big, zeros_1d,
                                      with_deg=(li == 0))
        agg_parts = agg_pad[:, :N_NODES, :]
        if li == 0:
            deg_parts = deg_out[:, :N_NODES].reshape(NC, N_NODES, 1)
        w1 = p["W_n1"]
        w1h = w1[:, :H]
        w1a = w1[:, H:]
        wea_next = params[li + 1]["W_ea"] if li + 1 < len(params) else None
        h_cur, hw = _node_update(
            h_cur, agg_parts, deg_parts, w1h, w1a, p["b_n1"], p["W_n2"],
            p["b_n2"], wea_next)
        e_cur = e_proj
    return (h_cur, e_cur)
